# Initial kernel scaffold; baseline (speedup 1.0000x reference)
import jax
import jax.numpy as jnp
from jax.experimental import pallas as pl


def kernel(U, edge_index, edge_values, W, B, W_init, V_w):
    raise NotImplementedError("write your pallas kernel here")


# speedup vs baseline: 9.3022x; 9.3022x over previous
import functools

import jax
import jax.numpy as jnp
from jax import lax
from jax.experimental import pallas as pl
from jax.experimental.pallas import tpu as pltpu
from jax.experimental.pallas import tpu_sc as plsc

N = 10000
E = 320000
F = 128
NCLASS = 64
KAPPA = 0.99
THRESHOLD = 30

NC = 2
NS = 16
NW = NC * NS

EPW = E // NW
CH = 80
NCHUNK = EPW // CH
NP2 = 10240
RPT = NP2 // NS

EPT = E // NS
PCH = EPT // CH
RSL = NP2 // NS

_BCAST_DNUMS = lax.GatherDimensionNumbers(
    offset_dims=(), collapsed_slice_dims=(0,), start_index_map=(0,))


def _lane_bcast(vec16, lane):
  idx = jnp.full((16, 1), lane, jnp.int32)
  return lax.gather(vec16, idx, _BCAST_DNUMS, (1,),
                    mode=lax.GatherScatterMode.PROMISE_IN_BOUNDS)


def _spmm_body(x_hbm, src_hbm, dst_hbm, val_hbm, out_hbm,
               rows_v, src_v, dst_v, val_v, acc_sh, sem):
  c = lax.axis_index("c")
  s = lax.axis_index("s")
  wid = c * NS + s

  z16 = jnp.zeros((16,), jnp.float32)
  def zrow(r, _):
    for j in range(F // 16):
      rows_v[r, pl.ds(j * 16, 16)] = z16
    return 0
  lax.fori_loop(0, CH, zrow, 0)
  def zcp(k, _):
    pltpu.sync_copy(rows_v, acc_sh.at[pl.ds(s * RPT + k * CH, CH)])
    return 0
  lax.fori_loop(0, RPT // CH, zcp, 0)
  plsc.subcore_barrier()

  def chunk_body(ch, _):
    base = wid * EPW + ch * CH
    pltpu.sync_copy(src_hbm.at[pl.ds(base, CH)], src_v)
    pltpu.sync_copy(dst_hbm.at[pl.ds(base, CH)], dst_v)
    pltpu.sync_copy(val_hbm.at[pl.ds(base, CH)], val_v)
    pltpu.async_copy(x_hbm.at[src_v], rows_v, sem).wait()
    def grp(g, _):
      vv = val_v[pl.ds(g * 16, 16)]
      for l in range(16):
        vb = _lane_bcast(vv, l)
        r = g * 16 + l
        for j in range(F // 16):
          rows_v[r, pl.ds(j * 16, 16)] = rows_v[r, pl.ds(j * 16, 16)] * vb
      return 0
    lax.fori_loop(0, CH // 16, grp, 0)
    pltpu.sync_copy(rows_v, acc_sh.at[dst_v], add=True)
    return 0
  lax.fori_loop(0, NCHUNK, chunk_body, 0)

  plsc.subcore_barrier()
  pltpu.sync_copy(acc_sh.at[pl.ds(s * RPT, RPT)],
                  out_hbm.at[c, pl.ds(s * RPT, RPT)])


@functools.partial(
    pl.kernel,
    out_type=jax.ShapeDtypeStruct((NC, NP2, F), jnp.float32),
    mesh=plsc.VectorSubcoreMesh(core_axis_name="c", subcore_axis_name="s"),
    scratch_types=[
        pltpu.VMEM((CH, F), jnp.float32),
        pltpu.VMEM((CH,), jnp.int32),
        pltpu.VMEM((CH,), jnp.int32),
        pltpu.VMEM((CH,), jnp.float32),
        pltpu.VMEM_SHARED((NP2, F), jnp.float32),
        pltpu.SemaphoreType.DMA,
    ],
)
def _spmm_sc(x_hbm, src_hbm, dst_hbm, val_hbm, out_hbm,
             rows_v, src_v, dst_v, val_v, acc_sh, sem):
  _spmm_body(x_hbm, src_hbm, dst_hbm, val_hbm, out_hbm,
             rows_v, src_v, dst_v, val_v, acc_sh, sem)


def _power_body(src_hbm, dst_hbm, val_hbm, v30_hbm, v31_hbm,
                src2, dst2, val2, gv, zb, cb, a_sh, b_sh, sem):
  c = lax.axis_index("c")
  s = lax.axis_index("s")

  pltpu.sync_copy(src_hbm.at[s], src2)
  pltpu.sync_copy(dst_hbm.at[s], dst2)
  pltpu.sync_copy(val_hbm.at[s], val2)

  z16 = jnp.zeros((16,), jnp.float32)
  c16 = jnp.full((16,), 1.0 / 100.0, jnp.float32)
  def init_bufs(i, _):
    zb[pl.ds(i * 16, 16)] = z16
    cb[pl.ds(i * 16, 16)] = jnp.where(s * RSL + i * 16 < N, c16, z16)
    return 0
  lax.fori_loop(0, RSL // 16, init_bufs, 0)
  pltpu.sync_copy(cb, a_sh.at[pl.ds(s * RSL, RSL)])
  plsc.subcore_barrier()

  def do_step(step, r_sh, w_sh):
    pltpu.sync_copy(zb, w_sh.at[pl.ds(s * RSL, RSL)])
    plsc.subcore_barrier()
    def chunk(ch, _):
      pltpu.async_copy(r_sh.at[src2.at[ch]], gv, sem).wait()
      def grp(g, _):
        gv[pl.ds(g * 16, 16)] = gv[pl.ds(g * 16, 16)] * val2[ch, pl.ds(g * 16, 16)]
        return 0
      lax.fori_loop(0, CH // 16, grp, 0)
      pltpu.sync_copy(gv, w_sh.at[dst2.at[ch]], add=True)
      return 0
    lax.fori_loop(0, PCH, chunk, 0)
    plsc.subcore_barrier()
    @pl.when((step == THRESHOLD - 1) & (c == 0))
    def _():
      pltpu.sync_copy(w_sh.at[pl.ds(s * RSL, RSL)],
                      v30_hbm.at[pl.ds(s * RSL, RSL)])
    @pl.when((step == THRESHOLD) & (c == 0))
    def _():
      pltpu.sync_copy(w_sh.at[pl.ds(s * RSL, RSL)],
                      v31_hbm.at[pl.ds(s * RSL, RSL)])

  def step_body(step, _):
    @pl.when(step % 2 == 0)
    def _():
      do_step(step, a_sh, b_sh)
    @pl.when(step % 2 == 1)
    def _():
      do_step(step, b_sh, a_sh)
    return 0
  lax.fori_loop(0, THRESHOLD + 1, step_body, 0)


@functools.partial(
    pl.kernel,
    out_type=(jax.ShapeDtypeStruct((NP2,), jnp.float32),
              jax.ShapeDtypeStruct((NP2,), jnp.float32)),
    mesh=plsc.VectorSubcoreMesh(core_axis_name="c", subcore_axis_name="s"),
    scratch_types=[
        pltpu.VMEM((PCH, CH), jnp.int32),
        pltpu.VMEM((PCH, CH), jnp.int32),
        pltpu.VMEM((PCH, CH), jnp.float32),
        pltpu.VMEM((CH,), jnp.float32),
        pltpu.VMEM((RSL,), jnp.float32),
        pltpu.VMEM((RSL,), jnp.float32),
        pltpu.VMEM_SHARED((NP2,), jnp.float32),
        pltpu.VMEM_SHARED((NP2,), jnp.float32),
        pltpu.SemaphoreType.DMA,
    ],
)
def _power_sc(src_hbm, dst_hbm, val_hbm, v30_hbm, v31_hbm,
              src2, dst2, val2, gv, zb, cb, a_sh, b_sh, sem):
  _power_body(src_hbm, dst_hbm, val_hbm, v30_hbm, v31_hbm,
              src2, dst2, val2, gv, zb, cb, a_sh, b_sh, sem)


def _proj_body(v30_ref, v31_ref, w_ref, out_ref):
  a30 = v30_ref[...]
  a31 = v31_ref[...]
  m = jnp.maximum(jnp.max(jnp.abs(a30)), 1e-30)
  a30 = a30 / m
  a31 = a31 / m
  rho = jnp.sqrt(jnp.sum(a31 * a31) / jnp.maximum(jnp.sum(a30 * a30), 1e-30))
  rho = jnp.maximum(rho, 1e-6)
  vrad = KAPPA / rho

  w = w_ref[...]
  absw = jnp.abs(w)
  rowsum = jnp.sum(absw, axis=1, keepdims=True)
  hi0 = jnp.max(absw, axis=1, keepdims=True)
  lo0 = jnp.zeros_like(hi0)
  def bis(i, carry):
    lo, hi = carry
    mid = 0.5 * (lo + hi)
    srow = jnp.sum(jnp.maximum(absw - mid, 0.0), axis=1, keepdims=True)
    pred = srow > vrad
    return (jnp.where(pred, mid, lo), jnp.where(pred, hi, mid))
  lo, hi = lax.fori_loop(0, 50, bis, (lo0, hi0))
  theta = 0.5 * (lo + hi)
  wproj = jnp.sign(w) * jnp.maximum(absw - theta, 0.0)
  out_ref[...] = jnp.where(rowsum > vrad, wproj, w)


def _proj_tc(v30, v31, w):
  return pl.pallas_call(
      _proj_body,
      out_shape=jax.ShapeDtypeStruct((F, F), jnp.float32),
  )(v30.reshape(100, 100), v31.reshape(100, 100), w)


BLK = 2000


def _init_body(au0_ref, au1_ref, u_ref, b_ref, wi_ref, aub_ref, z0_ref):
  au = au0_ref[...] + au1_ref[...]
  aub_ref[...] = jnp.dot(au, b_ref[...], preferred_element_type=jnp.float32,
                        precision=lax.Precision.HIGHEST)
  z0_ref[...] = jnp.dot(u_ref[...], wi_ref[...],
                        preferred_element_type=jnp.float32,
                        precision=lax.Precision.HIGHEST)


def _init_tc(au0, au1, u, b, w_init):
  row = pl.BlockSpec((BLK, F), lambda i: (i, 0))
  full = pl.BlockSpec((F, F), lambda i: (0, 0))
  return pl.pallas_call(
      _init_body,
      grid=(N // BLK,),
      in_specs=[row, row, row, full, full],
      out_specs=[row, row],
      out_shape=[jax.ShapeDtypeStruct((N, F), jnp.float32),
                 jax.ShapeDtypeStruct((N, F), jnp.float32)],
  )(au0, au1, u, b, w_init)


def _iter_body(s0_ref, s1_ref, wp_ref, aub_ref, z_ref):
  sm = s0_ref[...] + s1_ref[...]
  z = jnp.dot(sm, wp_ref[...], preferred_element_type=jnp.float32,
                        precision=lax.Precision.HIGHEST)
  z_ref[...] = jnp.maximum(z + aub_ref[...], 0.0)


def _iter_tc(s0, s1, wp, aub):
  row = pl.BlockSpec((BLK, F), lambda i: (i, 0))
  full = pl.BlockSpec((F, F), lambda i: (0, 0))
  return pl.pallas_call(
      _iter_body,
      grid=(N // BLK,),
      in_specs=[row, row, full, row],
      out_specs=row,
      out_shape=jax.ShapeDtypeStruct((N, F), jnp.float32),
  )(s0, s1, wp, aub)


def _out_body(z_ref, vt_ref, o_ref):
  o_ref[...] = jnp.dot(z_ref[...], vt_ref[...],
                       preferred_element_type=jnp.float32,
                        precision=lax.Precision.HIGHEST)


def _out_tc(z, vt):
  return pl.pallas_call(
      _out_body,
      grid=(N // BLK,),
      in_specs=[pl.BlockSpec((BLK, F), lambda i: (i, 0)),
                pl.BlockSpec((F, NCLASS), lambda i: (0, 0))],
      out_specs=pl.BlockSpec((BLK, NCLASS), lambda i: (i, 0)),
      out_shape=jax.ShapeDtypeStruct((N, NCLASS), jnp.float32),
  )(z, vt)


def kernel(U, edge_index, edge_values, W, B, W_init, V_w):
  dst = edge_index[0]
  src = edge_index[1]

  src3 = src.reshape(NS, PCH, CH)
  dst3 = dst.reshape(NS, PCH, CH)
  val3 = edge_values.reshape(NS, PCH, CH)

  v30p, v31p = _power_sc(src3, dst3, val3)
  wp = _proj_tc(v30p[:N], v31p[:N], W)

  au = _spmm_sc(U, src, dst, edge_values)
  aub, z = _init_tc(au[0, :N], au[1, :N], U, B, W_init)

  def body(i, z):
    s = _spmm_sc(z, src, dst, edge_values)
    return _iter_tc(s[0, :N], s[1, :N], wp, aub)
  z = lax.fori_loop(0, THRESHOLD, body, z)

  return _out_tc(z, V_w.T)
